# Initial kernel scaffold; baseline (speedup 1.0000x reference)
import jax
import jax.numpy as jnp
from jax.experimental import pallas as pl


def kernel(graph_embedding, edge_index, edge_weight, W_exp, b_exp, W1, b1, gamma, beta, running_mean, running_var, W2, b2):
    raise NotImplementedError("write your pallas kernel here")


# speedup vs baseline: 21.0106x; 21.0106x over previous
import functools

import jax
import jax.numpy as jnp
from jax import lax
from jax.experimental import pallas as pl
from jax.experimental.pallas import tpu as pltpu
from jax.experimental.pallas import tpu_sc as plsc

B = 100
EMB = 16
H0 = 32
H1 = 64
OUT_DIM = 3
NUM_NODES = 500
N = B * NUM_NODES
E = 800000

NC = 2
NS = 16
NW = NC * NS
CH = 128
KCH = 4
CPW = 200
NSUP = CPW // KCH
EPAD = NW * CPW * CH
NPAD = 51200
RPT = NPAD // NS
ZR = 400
RB = 2048
GRID = NPAD // RB

_f32 = jnp.float32
_i32 = jnp.int32


def _mesh():
    return plsc.VectorSubcoreMesh(core_axis_name="c", subcore_axis_name="s")


_SC_PARAMS = pltpu.CompilerParams(use_tc_tiling_on_sc=False)


def _sc_deg(d2, w2):
    @functools.partial(
        pl.kernel,
        out_type=jax.ShapeDtypeStruct((NC, NPAD), _f32),
        mesh=_mesh(),
        compiler_params=_SC_PARAMS,
        scratch_types=[
            pltpu.VMEM_SHARED((NPAD,), _f32),
            pltpu.VMEM((RPT,), _f32),
            pltpu.VMEM((KCH, CH), _i32),
            pltpu.VMEM((KCH, CH), _f32),
            pltpu.SemaphoreType.DMA,
        ],
    )
    def k(d_hbm, w_hbm, out, acc, zbuf, dbuf, wbuf, sem):
        c = lax.axis_index("c")
        s = lax.axis_index("s")
        wid = c * NS + s

        def zb(i, _):
            zbuf[pl.ds(i * 16, 16)] = jnp.zeros((16,), _f32)
            return 0

        lax.fori_loop(0, RPT // 16, zb, 0)
        pltpu.sync_copy(zbuf, acc.at[pl.ds(s * RPT, RPT)])
        plsc.subcore_barrier()

        def body(g, _):
            j = wid * NSUP + g
            pltpu.sync_copy(d_hbm.at[j], dbuf)
            pltpu.sync_copy(w_hbm.at[j], wbuf)
            descs = [
                pltpu.async_copy(wbuf.at[kk], acc.at[dbuf.at[kk]], sem, add=True)
                for kk in range(KCH)
            ]
            for dsc in descs:
                dsc.wait()
            return 0

        lax.fori_loop(0, NSUP, body, 0)
        plsc.subcore_barrier()
        pltpu.sync_copy(acc.at[pl.ds(s * RPT, RPT)], out.at[c, pl.ds(s * RPT, RPT)])

    return k(d2, w2)


def _sc_edge_agg(y, s2, d2, w2, width):
    @functools.partial(
        pl.kernel,
        out_type=jax.ShapeDtypeStruct((NC, NPAD, width), _f32),
        mesh=_mesh(),
        compiler_params=_SC_PARAMS,
        scratch_types=[
            pltpu.VMEM_SHARED((NPAD, width), _f32),
            pltpu.VMEM((KCH, CH), _i32),
            pltpu.VMEM((KCH, CH), _i32),
            pltpu.VMEM((KCH, CH), _f32),
            pltpu.VMEM((KCH, CH, width), _f32),
            pltpu.SemaphoreType.DMA,
        ],
    )
    def k(y_hbm, s_hbm, d_hbm, w_hbm, out, acc, sbuf, dbuf, wbuf, rows, sem):
        c = lax.axis_index("c")
        s = lax.axis_index("s")
        wid = c * NS + s

        def zb(i, _):
            for h in range(width // 16):
                rows[0, i, pl.ds(h * 16, 16)] = jnp.zeros((16,), _f32)
            return 0

        lax.fori_loop(0, CH, zb, 0)
        for r in range(RPT // CH):
            pltpu.sync_copy(rows.at[0], acc.at[pl.ds(s * RPT + r * CH, CH)])
        plsc.subcore_barrier()

        def body(g, _):
            j = wid * NSUP + g
            pltpu.sync_copy(s_hbm.at[j], sbuf)
            pltpu.sync_copy(d_hbm.at[j], dbuf)
            pltpu.sync_copy(w_hbm.at[j], wbuf)
            gd = [
                pltpu.async_copy(y_hbm.at[sbuf.at[kk]], rows.at[kk], sem)
                for kk in range(KCH)
            ]
            for dsc in gd:
                dsc.wait()
            for kk in range(KCH):
                def sc_body(q, _):
                    wv16 = wbuf[kk, pl.ds(q * 16, 16)]
                    for j2 in range(16):
                        wv = wv16[j2]
                        for h in range(width // 16):
                            rows[kk, q * 16 + j2, pl.ds(h * 16, 16)] = (
                                rows[kk, q * 16 + j2, pl.ds(h * 16, 16)] * wv
                            )
                    return 0

                lax.fori_loop(0, CH // 16, sc_body, 0)
            sd = [
                pltpu.async_copy(rows.at[kk], acc.at[dbuf.at[kk]], sem, add=True)
                for kk in range(KCH)
            ]
            for dsc in sd:
                dsc.wait()
            return 0

        lax.fori_loop(0, NSUP, body, 0)
        plsc.subcore_barrier()
        pltpu.sync_copy(
            acc.at[pl.ds(s * RPT, RPT)], out.at[c, pl.ds(s * RPT, RPT), :]
        )

    return k(y, s2, d2, w2)


def _tc_expand(g, w_exp, b_exp):
    def body(g_ref, w_ref, b_ref, o_ref):
        o_ref[...] = (
            lax.dot_general(
                g_ref[...], w_ref[...], (((1,), (1,)), ((), ())),
                preferred_element_type=_f32,
            )
            + b_ref[...]
        )

    return pl.pallas_call(
        body,
        out_shape=jax.ShapeDtypeStruct((B, NUM_NODES * H0), _f32),
    )(g, w_exp, b_exp)


def _tc_prep(x0, da, db):
    def body(x_ref, a_ref, b_ref, y_ref, d_ref):
        deg = 1.0 + a_ref[...] + b_ref[...]
        dis = lax.rsqrt(deg)
        d_ref[...] = dis
        y_ref[...] = dis * x_ref[...]

    return pl.pallas_call(
        body,
        grid=(GRID,),
        in_specs=[
            pl.BlockSpec((RB, H0), lambda i: (i, 0)),
            pl.BlockSpec((RB, 1), lambda i: (i, 0)),
            pl.BlockSpec((RB, 1), lambda i: (i, 0)),
        ],
        out_specs=[
            pl.BlockSpec((RB, H0), lambda i: (i, 0)),
            pl.BlockSpec((RB, 1), lambda i: (i, 0)),
        ],
        out_shape=[
            jax.ShapeDtypeStruct((NPAD, H0), _f32),
            jax.ShapeDtypeStruct((NPAD, 1), _f32),
        ],
    )(x0, da, db)


def _tc_mid(a0, a1, y0, dis, w1, b1, gamma, beta, mean, var, w2p):
    def body(a0_ref, a1_ref, y_ref, d_ref, w1_ref, b1_ref, g_ref, be_ref,
             m_ref, v_ref, w2_ref, o_ref):
        t = d_ref[...] * (a0_ref[...] + a1_ref[...] + y_ref[...])
        o1 = lax.dot_general(
            t, w1_ref[...], (((1,), (1,)), ((), ())), preferred_element_type=_f32
        ) + b1_ref[...]
        sc = g_ref[...] * lax.rsqrt(v_ref[...] + 1e-5)
        x1 = jnp.maximum((o1 - m_ref[...]) * sc + be_ref[...], 0.0)
        h2 = lax.dot_general(
            x1, w2_ref[...], (((1,), (1,)), ((), ())), preferred_element_type=_f32
        )
        o_ref[...] = d_ref[...] * h2

    rowspec = lambda w: pl.BlockSpec((RB, w), lambda i: (i, 0))
    full = lambda r, w: pl.BlockSpec((r, w), lambda i: (0, 0))
    return pl.pallas_call(
        body,
        grid=(GRID,),
        in_specs=[
            rowspec(H0), rowspec(H0), rowspec(H0), rowspec(1),
            full(H1, H0), full(1, H1), full(1, H1), full(1, H1),
            full(1, H1), full(1, H1), full(16, H1),
        ],
        out_specs=rowspec(16),
        out_shape=jax.ShapeDtypeStruct((NPAD, 16), _f32),
    )(a0, a1, y0, dis, w1, b1, gamma, beta, mean, var, w2p)


def _tc_final(p0, p1, y2, dis, b2p):
    def body(p0_ref, p1_ref, y_ref, d_ref, b_ref, o_ref):
        o_ref[...] = d_ref[...] * (p0_ref[...] + p1_ref[...] + y_ref[...]) + b_ref[...]

    rowspec = lambda w: pl.BlockSpec((RB, w), lambda i: (i, 0))
    return pl.pallas_call(
        body,
        grid=(GRID,),
        in_specs=[
            rowspec(16), rowspec(16), rowspec(16), rowspec(1),
            pl.BlockSpec((1, 16), lambda i: (0, 0)),
        ],
        out_specs=rowspec(16),
        out_shape=jax.ShapeDtypeStruct((NPAD, 16), _f32),
    )(p0, p1, y2, dis, b2p)


def kernel(graph_embedding, edge_index, edge_weight, W_exp, b_exp, W1, b1,
           gamma, beta, running_mean, running_var, W2, b2):
    s32 = edge_index[0].astype(_i32)
    d32 = edge_index[1].astype(_i32)
    w = edge_weight.astype(_f32)
    pad = EPAD - E
    esh = (NW * NSUP, KCH, CH)
    s2 = jnp.concatenate([s32, jnp.zeros((pad,), _i32)]).reshape(esh)
    d2 = jnp.concatenate([d32, jnp.zeros((pad,), _i32)]).reshape(esh)
    w2 = jnp.concatenate([w, jnp.zeros((pad,), _f32)]).reshape(esh)

    degp = _sc_deg(d2, w2)
    x0f = _tc_expand(graph_embedding, W_exp, b_exp.reshape(1, -1))
    x0 = jnp.pad(x0f.reshape(N, H0), ((0, NPAD - N), (0, 0)))
    y0, dis = _tc_prep(x0, degp[0].reshape(NPAD, 1), degp[1].reshape(NPAD, 1))
    agg1 = _sc_edge_agg(y0, s2, d2, w2, H0)
    w2p = jnp.pad(W2, ((0, 16 - OUT_DIM), (0, 0)))
    y2 = _tc_mid(agg1[0], agg1[1], y0, dis, W1, b1.reshape(1, -1),
                 gamma.reshape(1, -1), beta.reshape(1, -1),
                 running_mean.reshape(1, -1), running_var.reshape(1, -1), w2p)
    agg2 = _sc_edge_agg(y2, s2, d2, w2, 16)
    b2p = jnp.pad(b2, (0, 16 - OUT_DIM)).reshape(1, 16)
    o = _tc_final(agg2[0], agg2[1], y2, dis, b2p)
    return o[:N, :OUT_DIM]


# speedup vs baseline: 26.8530x; 1.2781x over previous
import functools

import jax
import jax.numpy as jnp
from jax import lax
from jax.experimental import pallas as pl
from jax.experimental.pallas import tpu as pltpu
from jax.experimental.pallas import tpu_sc as plsc

B = 100
EMB = 16
H0 = 32
H1 = 64
OUT_DIM = 3
NUM_NODES = 500
N = B * NUM_NODES
E = 800000

NC = 2
NS = 16
NW = NC * NS
CH = 128
KCH = 2
CPW = 200
NSUP = CPW // KCH
EPAD = NW * CPW * CH
NPAD = 51200
RPT = NPAD // NS
ZR = 400
RB = 2048
GRID = NPAD // RB

_f32 = jnp.float32
_i32 = jnp.int32


def _mesh():
    return plsc.VectorSubcoreMesh(core_axis_name="c", subcore_axis_name="s")


_SC_PARAMS = pltpu.CompilerParams(use_tc_tiling_on_sc=False)


def _sc_deg(d2, w2):
    @functools.partial(
        pl.kernel,
        out_type=jax.ShapeDtypeStruct((NC, NPAD), _f32),
        mesh=_mesh(),
        compiler_params=_SC_PARAMS,
        scratch_types=[
            pltpu.VMEM_SHARED((NPAD,), _f32),
            pltpu.VMEM((RPT,), _f32),
            pltpu.VMEM((3, KCH, CH), _i32),
            pltpu.VMEM((3, KCH, CH), _f32),
            pltpu.SemaphoreType.DMA,
            pltpu.SemaphoreType.DMA,
        ],
    )
    def k(d_hbm, w_hbm, out, acc, zbuf, dbuf, wbuf, sem_e, sem_s):
        c = lax.axis_index("c")
        s = lax.axis_index("s")
        wid = c * NS + s

        def zb(i, _):
            zbuf[pl.ds(i * 16, 16)] = jnp.zeros((16,), _f32)
            return 0

        lax.fori_loop(0, RPT // 16, zb, 0)
        pltpu.sync_copy(zbuf, acc.at[pl.ds(s * RPT, RPT)])
        plsc.subcore_barrier()

        def issue_edge(j, b):
            pltpu.async_copy(d_hbm.at[j], dbuf.at[b], sem_e)
            pltpu.async_copy(w_hbm.at[j], wbuf.at[b], sem_e)

        def wait_edge(b):
            pltpu.make_async_copy(d_hbm.at[0], dbuf.at[b], sem_e).wait()
            pltpu.make_async_copy(w_hbm.at[0], wbuf.at[b], sem_e).wait()

        def issue_scatter(b):
            for kk in range(KCH):
                pltpu.async_copy(
                    wbuf.at[b, kk], acc.at[dbuf.at[b, kk]], sem_s, add=True
                )

        def wait_scatter(b):
            for kk in range(KCH):
                pltpu.make_async_copy(
                    wbuf.at[b, kk], acc.at[dbuf.at[b, kk]], sem_s
                ).wait()

        j0 = wid * NSUP
        issue_edge(j0, 0)
        wait_edge(0)
        issue_scatter(0)
        issue_edge(j0 + 1, 1)
        wait_edge(1)
        issue_scatter(1)
        issue_edge(j0 + 2, 2)

        def body(t, _):
            g = 2 + 3 * t
            for (cur, prv, nxt), dg in (((2, 1, 0), 0), ((0, 2, 1), 1),
                                        ((1, 0, 2), 2)):
                wait_edge(cur)
                wait_scatter(nxt)
                issue_scatter(cur)
                issue_edge(j0 + g + dg + 1, nxt)
            return 0

        lax.fori_loop(0, (NSUP - 4) // 3, body, 0)
        wait_edge(2)
        wait_scatter(0)
        issue_scatter(2)
        issue_edge(j0 + NSUP - 1, 0)
        wait_edge(0)
        wait_scatter(1)
        issue_scatter(0)
        wait_scatter(2)
        wait_scatter(0)
        plsc.subcore_barrier()
        pltpu.sync_copy(acc.at[pl.ds(s * RPT, RPT)], out.at[c, pl.ds(s * RPT, RPT)])

    return k(d2, w2)


def _sc_edge_agg(y, s2, d2, w2, width):
    @functools.partial(
        pl.kernel,
        out_type=jax.ShapeDtypeStruct((NC, NPAD, width), _f32),
        mesh=_mesh(),
        compiler_params=_SC_PARAMS,
        scratch_types=[
            pltpu.VMEM_SHARED((NPAD, width), _f32),
            pltpu.VMEM((3, KCH, CH), _i32),
            pltpu.VMEM((3, KCH, CH), _i32),
            pltpu.VMEM((3, KCH, CH), _f32),
            pltpu.VMEM((3, KCH, CH, width), _f32),
            pltpu.SemaphoreType.DMA,
            pltpu.SemaphoreType.DMA,
            pltpu.SemaphoreType.DMA,
        ],
    )
    def k(y_hbm, s_hbm, d_hbm, w_hbm, out, acc, sbuf, dbuf, wbuf, rows,
          sem_e, sem_g, sem_s):
        c = lax.axis_index("c")
        s = lax.axis_index("s")
        wid = c * NS + s

        def zb(i, _):
            for h in range(width // 16):
                rows[0, 0, i, pl.ds(h * 16, 16)] = jnp.zeros((16,), _f32)
            return 0

        lax.fori_loop(0, CH, zb, 0)
        for r in range(RPT // CH):
            pltpu.sync_copy(rows.at[0, 0], acc.at[pl.ds(s * RPT + r * CH, CH)])
        plsc.subcore_barrier()

        def issue_edge(j, b):
            pltpu.async_copy(s_hbm.at[j], sbuf.at[b], sem_e)
            pltpu.async_copy(d_hbm.at[j], dbuf.at[b], sem_e)
            pltpu.async_copy(w_hbm.at[j], wbuf.at[b], sem_e)

        def wait_edge(b):
            pltpu.make_async_copy(s_hbm.at[0], sbuf.at[b], sem_e).wait()
            pltpu.make_async_copy(d_hbm.at[0], dbuf.at[b], sem_e).wait()
            pltpu.make_async_copy(w_hbm.at[0], wbuf.at[b], sem_e).wait()

        def issue_gather(b):
            for kk in range(KCH):
                pltpu.async_copy(y_hbm.at[sbuf.at[b, kk]], rows.at[b, kk], sem_g)

        def wait_gather(b):
            for kk in range(KCH):
                pltpu.make_async_copy(
                    y_hbm.at[sbuf.at[b, kk]], rows.at[b, kk], sem_g
                ).wait()

        def scale(b):
            for kk in range(KCH):
                def sc_body(q, _):
                    wv16 = wbuf[b, kk, pl.ds(q * 16, 16)]
                    for j2 in range(16):
                        wv = wv16[j2]
                        for h in range(width // 16):
                            rows[b, kk, q * 16 + j2, pl.ds(h * 16, 16)] = (
                                rows[b, kk, q * 16 + j2, pl.ds(h * 16, 16)] * wv
                            )
                    return 0

                lax.fori_loop(0, CH // 16, sc_body, 0)

        def issue_scatter(b):
            for kk in range(KCH):
                pltpu.async_copy(
                    rows.at[b, kk], acc.at[dbuf.at[b, kk]], sem_s, add=True
                )

        def wait_scatter(b):
            for kk in range(KCH):
                pltpu.make_async_copy(
                    rows.at[b, kk], acc.at[dbuf.at[b, kk]], sem_s
                ).wait()

        j0 = wid * NSUP
        issue_edge(j0, 0)
        wait_edge(0)
        issue_gather(0)
        issue_edge(j0 + 1, 1)
        wait_edge(1)
        issue_gather(1)
        issue_edge(j0 + 2, 2)
        wait_gather(0)
        scale(0)
        issue_scatter(0)

        def body(t, _):
            g = 2 + 3 * t
            for (cur, prv, nxt), dg in (((2, 1, 0), 0), ((0, 2, 1), 1),
                                        ((1, 0, 2), 2)):
                wait_edge(cur)
                wait_scatter(nxt)
                issue_gather(cur)
                issue_edge(j0 + g + dg + 1, nxt)
                wait_gather(prv)
                scale(prv)
                issue_scatter(prv)
            return 0

        lax.fori_loop(0, (NSUP - 4) // 3, body, 0)
        wait_edge(2)
        wait_scatter(0)
        issue_gather(2)
        issue_edge(j0 + NSUP - 1, 0)
        wait_gather(1)
        scale(1)
        issue_scatter(1)
        wait_edge(0)
        wait_scatter(1)
        issue_gather(0)
        wait_gather(2)
        scale(2)
        issue_scatter(2)
        wait_gather(0)
        scale(0)
        issue_scatter(0)
        wait_scatter(2)
        wait_scatter(0)
        plsc.subcore_barrier()
        pltpu.sync_copy(
            acc.at[pl.ds(s * RPT, RPT)], out.at[c, pl.ds(s * RPT, RPT), :]
        )

    return k(y, s2, d2, w2)


def _tc_expand(g, w_exp, b_exp):
    def body(g_ref, w_ref, b_ref, o_ref):
        o_ref[...] = (
            lax.dot_general(
                g_ref[...], w_ref[...], (((1,), (1,)), ((), ())),
                preferred_element_type=_f32,
            )
            + b_ref[...]
        )

    return pl.pallas_call(
        body,
        out_shape=jax.ShapeDtypeStruct((B, NUM_NODES * H0), _f32),
    )(g, w_exp, b_exp)


def _tc_prep(x0, da, db):
    def body(x_ref, a_ref, b_ref, y_ref, d_ref):
        deg = 1.0 + a_ref[...] + b_ref[...]
        dis = lax.rsqrt(deg)
        d_ref[...] = dis
        y_ref[...] = dis * x_ref[...]

    return pl.pallas_call(
        body,
        grid=(GRID,),
        in_specs=[
            pl.BlockSpec((RB, H0), lambda i: (i, 0)),
            pl.BlockSpec((RB, 1), lambda i: (i, 0)),
            pl.BlockSpec((RB, 1), lambda i: (i, 0)),
        ],
        out_specs=[
            pl.BlockSpec((RB, H0), lambda i: (i, 0)),
            pl.BlockSpec((RB, 1), lambda i: (i, 0)),
        ],
        out_shape=[
            jax.ShapeDtypeStruct((NPAD, H0), _f32),
            jax.ShapeDtypeStruct((NPAD, 1), _f32),
        ],
    )(x0, da, db)


def _tc_mid(a0, a1, y0, dis, w1, b1, gamma, beta, mean, var, w2p):
    def body(a0_ref, a1_ref, y_ref, d_ref, w1_ref, b1_ref, g_ref, be_ref,
             m_ref, v_ref, w2_ref, o_ref):
        t = d_ref[...] * (a0_ref[...] + a1_ref[...] + y_ref[...])
        o1 = lax.dot_general(
            t, w1_ref[...], (((1,), (1,)), ((), ())), preferred_element_type=_f32
        ) + b1_ref[...]
        sc = g_ref[...] * lax.rsqrt(v_ref[...] + 1e-5)
        x1 = jnp.maximum((o1 - m_ref[...]) * sc + be_ref[...], 0.0)
        h2 = lax.dot_general(
            x1, w2_ref[...], (((1,), (1,)), ((), ())), preferred_element_type=_f32
        )
        o_ref[...] = d_ref[...] * h2

    rowspec = lambda w: pl.BlockSpec((RB, w), lambda i: (i, 0))
    full = lambda r, w: pl.BlockSpec((r, w), lambda i: (0, 0))
    return pl.pallas_call(
        body,
        grid=(GRID,),
        in_specs=[
            rowspec(H0), rowspec(H0), rowspec(H0), rowspec(1),
            full(H1, H0), full(1, H1), full(1, H1), full(1, H1),
            full(1, H1), full(1, H1), full(16, H1),
        ],
        out_specs=rowspec(16),
        out_shape=jax.ShapeDtypeStruct((NPAD, 16), _f32),
    )(a0, a1, y0, dis, w1, b1, gamma, beta, mean, var, w2p)


def _tc_final(p0, p1, y2, dis, b2p):
    def body(p0_ref, p1_ref, y_ref, d_ref, b_ref, o_ref):
        o_ref[...] = d_ref[...] * (p0_ref[...] + p1_ref[...] + y_ref[...]) + b_ref[...]

    rowspec = lambda w: pl.BlockSpec((RB, w), lambda i: (i, 0))
    return pl.pallas_call(
        body,
        grid=(GRID,),
        in_specs=[
            rowspec(16), rowspec(16), rowspec(16), rowspec(1),
            pl.BlockSpec((1, 16), lambda i: (0, 0)),
        ],
        out_specs=rowspec(16),
        out_shape=jax.ShapeDtypeStruct((NPAD, 16), _f32),
    )(p0, p1, y2, dis, b2p)


def kernel(graph_embedding, edge_index, edge_weight, W_exp, b_exp, W1, b1,
           gamma, beta, running_mean, running_var, W2, b2):
    s32 = edge_index[0].astype(_i32)
    d32 = edge_index[1].astype(_i32)
    w = edge_weight.astype(_f32)
    pad = EPAD - E
    esh = (NW * NSUP, KCH, CH)
    s2 = jnp.concatenate([s32, jnp.zeros((pad,), _i32)]).reshape(esh)
    d2 = jnp.concatenate([d32, jnp.zeros((pad,), _i32)]).reshape(esh)
    w2 = jnp.concatenate([w, jnp.zeros((pad,), _f32)]).reshape(esh)

    degp = _sc_deg(d2, w2)
    x0f = _tc_expand(graph_embedding, W_exp, b_exp.reshape(1, -1))
    x0 = jnp.pad(x0f.reshape(N, H0), ((0, NPAD - N), (0, 0)))
    y0, dis = _tc_prep(x0, degp[0].reshape(NPAD, 1), degp[1].reshape(NPAD, 1))
    agg1 = _sc_edge_agg(y0, s2, d2, w2, H0)
    w2p = jnp.pad(W2, ((0, 16 - OUT_DIM), (0, 0)))
    y2 = _tc_mid(agg1[0], agg1[1], y0, dis, W1, b1.reshape(1, -1),
                 gamma.reshape(1, -1), beta.reshape(1, -1),
                 running_mean.reshape(1, -1), running_var.reshape(1, -1), w2p)
    agg2 = _sc_edge_agg(y2, s2, d2, w2, 16)
    b2p = jnp.pad(b2, (0, 16 - OUT_DIM)).reshape(1, 16)
    o = _tc_final(agg2[0], agg2[1], y2, dis, b2p)
    return o[:N, :OUT_DIM]


# speedup vs baseline: 36.2641x; 1.3505x over previous
import functools

import jax
import jax.numpy as jnp
from jax import lax
from jax.experimental import pallas as pl
from jax.experimental.pallas import tpu as pltpu
from jax.experimental.pallas import tpu_sc as plsc

B = 100
EMB = 16
H0 = 32
H1 = 64
OUT_DIM = 3
NUM_NODES = 500
N = B * NUM_NODES
E = 800000

NC = 2
NS = 16
NW = NC * NS
CH = 128
KCH = 2
CPW = 200
NSUP = CPW // KCH
EPAD = NW * CPW * CH
NPAD = 51200
RPT = NPAD // NS
ZR = 400
RB = 2048
GRID = NPAD // RB

_f32 = jnp.float32
_i32 = jnp.int32


def _mesh():
    return plsc.VectorSubcoreMesh(core_axis_name="c", subcore_axis_name="s")


_SC_PARAMS = pltpu.CompilerParams(use_tc_tiling_on_sc=False)


def _sc_deg(d2, w2):
    @functools.partial(
        pl.kernel,
        out_type=jax.ShapeDtypeStruct((NC, NPAD), _f32),
        mesh=_mesh(),
        compiler_params=_SC_PARAMS,
        scratch_types=[
            pltpu.VMEM_SHARED((NPAD,), _f32),
            pltpu.VMEM((RPT,), _f32),
            pltpu.VMEM((3, KCH, CH), _i32),
            pltpu.VMEM((3, KCH, CH), _f32),
            pltpu.SemaphoreType.DMA,
            pltpu.SemaphoreType.DMA,
        ],
    )
    def k(d_hbm, w_hbm, out, acc, zbuf, dbuf, wbuf, sem_e, sem_s):
        c = lax.axis_index("c")
        s = lax.axis_index("s")
        wid = c * NS + s

        def zb(i, _):
            zbuf[pl.ds(i * 16, 16)] = jnp.zeros((16,), _f32)
            return 0

        lax.fori_loop(0, RPT // 16, zb, 0)
        pltpu.sync_copy(zbuf, acc.at[pl.ds(s * RPT, RPT)])
        plsc.subcore_barrier()

        def issue_edge(j, b):
            pltpu.async_copy(d_hbm.at[j], dbuf.at[b], sem_e)
            pltpu.async_copy(w_hbm.at[j], wbuf.at[b], sem_e)

        def wait_edge(b):
            pltpu.make_async_copy(d_hbm.at[0], dbuf.at[b], sem_e).wait()
            pltpu.make_async_copy(w_hbm.at[0], wbuf.at[b], sem_e).wait()

        def issue_scatter(b):
            for kk in range(KCH):
                pltpu.async_copy(
                    wbuf.at[b, kk], acc.at[dbuf.at[b, kk]], sem_s, add=True
                )

        def wait_scatter(b):
            for kk in range(KCH):
                pltpu.make_async_copy(
                    wbuf.at[b, kk], acc.at[dbuf.at[b, kk]], sem_s
                ).wait()

        j0 = wid * NSUP
        issue_edge(j0, 0)
        wait_edge(0)
        issue_scatter(0)
        issue_edge(j0 + 1, 1)
        wait_edge(1)
        issue_scatter(1)
        issue_edge(j0 + 2, 2)

        def body(t, _):
            g = 2 + 3 * t
            for (cur, prv, nxt), dg in (((2, 1, 0), 0), ((0, 2, 1), 1),
                                        ((1, 0, 2), 2)):
                wait_edge(cur)
                wait_scatter(nxt)
                issue_scatter(cur)
                issue_edge(j0 + g + dg + 1, nxt)
            return 0

        lax.fori_loop(0, (NSUP - 4) // 3, body, 0)
        wait_edge(2)
        wait_scatter(0)
        issue_scatter(2)
        issue_edge(j0 + NSUP - 1, 0)
        wait_edge(0)
        wait_scatter(1)
        issue_scatter(0)
        wait_scatter(2)
        wait_scatter(0)
        plsc.subcore_barrier()
        pltpu.sync_copy(acc.at[pl.ds(s * RPT, RPT)], out.at[c, pl.ds(s * RPT, RPT)])

    return k(d2, w2)


def _sc_edge_agg(y, s2, d2, w2, width):
    @functools.partial(
        pl.kernel,
        out_type=jax.ShapeDtypeStruct((NC, NPAD, width), _f32),
        mesh=_mesh(),
        compiler_params=_SC_PARAMS,
        scratch_types=[
            pltpu.VMEM_SHARED((NPAD, width), _f32),
            pltpu.VMEM((3, KCH, CH), _i32),
            pltpu.VMEM((3, KCH, CH), _i32),
            pltpu.VMEM((3, KCH, CH), _f32),
            pltpu.VMEM((3, KCH, CH, width), _f32),
            pltpu.SemaphoreType.DMA,
            pltpu.SemaphoreType.DMA,
            pltpu.SemaphoreType.DMA,
        ],
    )
    def k(y_hbm, s_hbm, d_hbm, w_hbm, out, acc, sbuf, dbuf, wbuf, rows,
          sem_e, sem_g, sem_s):
        c = lax.axis_index("c")
        s = lax.axis_index("s")
        wid = c * NS + s

        def zb(i, _):
            for h in range(width // 16):
                rows[0, 0, i, pl.ds(h * 16, 16)] = jnp.zeros((16,), _f32)
            return 0

        lax.fori_loop(0, CH, zb, 0)
        for r in range(RPT // CH):
            pltpu.sync_copy(rows.at[0, 0], acc.at[pl.ds(s * RPT + r * CH, CH)])
        plsc.subcore_barrier()

        def issue_edge(j, b):
            pltpu.async_copy(s_hbm.at[j], sbuf.at[b], sem_e)
            pltpu.async_copy(d_hbm.at[j], dbuf.at[b], sem_e)
            pltpu.async_copy(w_hbm.at[j], wbuf.at[b], sem_e)

        def wait_edge(b):
            pltpu.make_async_copy(s_hbm.at[0], sbuf.at[b], sem_e).wait()
            pltpu.make_async_copy(d_hbm.at[0], dbuf.at[b], sem_e).wait()
            pltpu.make_async_copy(w_hbm.at[0], wbuf.at[b], sem_e).wait()

        def issue_gather(b):
            for kk in range(KCH):
                pltpu.async_copy(y_hbm.at[sbuf.at[b, kk]], rows.at[b, kk], sem_g)

        def wait_gather(b):
            for kk in range(KCH):
                pltpu.make_async_copy(
                    y_hbm.at[sbuf.at[b, kk]], rows.at[b, kk], sem_g
                ).wait()

        def scale(b):
            for kk in range(KCH):
                def sc_body(q, _):
                    wv16 = wbuf[b, kk, pl.ds(q * 16, 16)]
                    for j2 in range(16):
                        wv = wv16[j2]
                        for h in range(width // 16):
                            rows[b, kk, q * 16 + j2, pl.ds(h * 16, 16)] = (
                                rows[b, kk, q * 16 + j2, pl.ds(h * 16, 16)] * wv
                            )
                    return 0

                lax.fori_loop(0, CH // 16, sc_body, 0)

        def issue_scatter(b):
            for kk in range(KCH):
                pltpu.async_copy(
                    rows.at[b, kk], acc.at[dbuf.at[b, kk]], sem_s, add=True
                )

        def wait_scatter(b):
            for kk in range(KCH):
                pltpu.make_async_copy(
                    rows.at[b, kk], acc.at[dbuf.at[b, kk]], sem_s
                ).wait()

        j0 = wid * NSUP
        issue_edge(j0, 0)
        wait_edge(0)
        issue_gather(0)
        issue_edge(j0 + 1, 1)
        wait_edge(1)
        issue_gather(1)
        issue_edge(j0 + 2, 2)
        wait_gather(0)
        scale(0)
        issue_scatter(0)

        def body(t, _):
            g = 2 + 3 * t
            for (cur, prv, nxt), dg in (((2, 1, 0), 0), ((0, 2, 1), 1),
                                        ((1, 0, 2), 2)):
                wait_edge(cur)
                wait_scatter(nxt)
                issue_gather(cur)
                issue_edge(j0 + g + dg + 1, nxt)
                wait_gather(prv)
                scale(prv)
                issue_scatter(prv)
            return 0

        lax.fori_loop(0, (NSUP - 4) // 3, body, 0)
        wait_edge(2)
        wait_scatter(0)
        issue_gather(2)
        issue_edge(j0 + NSUP - 1, 0)
        wait_gather(1)
        scale(1)
        issue_scatter(1)
        wait_edge(0)
        wait_scatter(1)
        issue_gather(0)
        wait_gather(2)
        scale(2)
        issue_scatter(2)
        wait_gather(0)
        scale(0)
        issue_scatter(0)
        wait_scatter(2)
        wait_scatter(0)
        plsc.subcore_barrier()
        pltpu.sync_copy(
            acc.at[pl.ds(s * RPT, RPT)], out.at[c, pl.ds(s * RPT, RPT), :]
        )

    return k(y, s2, d2, w2)


def _tc_expand(g, w_exp, b_exp):
    def body(g_ref, w_ref, b_ref, o_ref):
        o_ref[...] = (
            lax.dot_general(
                g_ref[...], w_ref[...], (((1,), (1,)), ((), ())),
                preferred_element_type=_f32,
            )
            + b_ref[...]
        )

    return pl.pallas_call(
        body,
        out_shape=jax.ShapeDtypeStruct((B, NUM_NODES * H0), _f32),
    )(g, w_exp, b_exp)


def _tc_prep(x0, da, db):
    def body(x_ref, a_ref, b_ref, y_ref, d_ref):
        deg = 1.0 + a_ref[...] + b_ref[...]
        dis = lax.rsqrt(deg)
        d_ref[...] = dis
        y_ref[...] = dis * x_ref[...]

    return pl.pallas_call(
        body,
        grid=(GRID,),
        in_specs=[
            pl.BlockSpec((RB, H0), lambda i: (i, 0)),
            pl.BlockSpec((RB, 1), lambda i: (i, 0)),
            pl.BlockSpec((RB, 1), lambda i: (i, 0)),
        ],
        out_specs=[
            pl.BlockSpec((RB, H0), lambda i: (i, 0)),
            pl.BlockSpec((RB, 1), lambda i: (i, 0)),
        ],
        out_shape=[
            jax.ShapeDtypeStruct((NPAD, H0), _f32),
            jax.ShapeDtypeStruct((NPAD, 1), _f32),
        ],
    )(x0, da, db)


def _tc_mid(a0, a1, y0, dis, w1, b1, gamma, beta, mean, var, w2p):
    def body(a0_ref, a1_ref, y_ref, d_ref, w1_ref, b1_ref, g_ref, be_ref,
             m_ref, v_ref, w2_ref, o_ref):
        t = d_ref[...] * (a0_ref[...] + a1_ref[...] + y_ref[...])
        o1 = lax.dot_general(
            t, w1_ref[...], (((1,), (1,)), ((), ())), preferred_element_type=_f32
        ) + b1_ref[...]
        sc = g_ref[...] * lax.rsqrt(v_ref[...] + 1e-5)
        x1 = jnp.maximum((o1 - m_ref[...]) * sc + be_ref[...], 0.0)
        h2 = lax.dot_general(
            x1, w2_ref[...], (((1,), (1,)), ((), ())), preferred_element_type=_f32
        )
        o_ref[...] = d_ref[...] * h2

    rowspec = lambda w: pl.BlockSpec((RB, w), lambda i: (i, 0))
    full = lambda r, w: pl.BlockSpec((r, w), lambda i: (0, 0))
    return pl.pallas_call(
        body,
        grid=(GRID,),
        in_specs=[
            rowspec(H0), rowspec(H0), rowspec(H0), rowspec(1),
            full(H1, H0), full(1, H1), full(1, H1), full(1, H1),
            full(1, H1), full(1, H1), full(16, H1),
        ],
        out_specs=rowspec(16),
        out_shape=jax.ShapeDtypeStruct((NPAD, 16), _f32),
    )(a0, a1, y0, dis, w1, b1, gamma, beta, mean, var, w2p)


def _tc_final(p0, p1, y2, dis, b2p):
    def body(p0_ref, p1_ref, y_ref, d_ref, b_ref, o_ref):
        o = d_ref[...] * (p0_ref[...] + p1_ref[...] + y_ref[...]) + b_ref[...]
        o_ref[...] = o[:, :OUT_DIM]

    RF = N // GRID
    rowspec = lambda w: pl.BlockSpec((RF, w), lambda i: (i, 0))
    return pl.pallas_call(
        body,
        grid=(GRID,),
        in_specs=[
            rowspec(16), rowspec(16), rowspec(16), rowspec(1),
            pl.BlockSpec((1, 16), lambda i: (0, 0)),
        ],
        out_specs=rowspec(OUT_DIM),
        out_shape=jax.ShapeDtypeStruct((N, OUT_DIM), _f32),
    )(p0, p1, y2, dis, b2p)


def kernel(graph_embedding, edge_index, edge_weight, W_exp, b_exp, W1, b1,
           gamma, beta, running_mean, running_var, W2, b2):
    s32 = edge_index[0].astype(_i32)
    d32 = edge_index[1].astype(_i32)
    w = edge_weight.astype(_f32)
    pad = EPAD - E
    esh = (NW * NSUP, KCH, CH)
    padidx = N + (jnp.arange(pad, dtype=_i32) % (NPAD - N))
    s2 = jnp.concatenate([s32, padidx]).reshape(esh)
    d2 = jnp.concatenate([d32, padidx]).reshape(esh)
    w2 = jnp.concatenate([w, jnp.zeros((pad,), _f32)]).reshape(esh)

    degp = _sc_deg(d2, w2)
    x0f = _tc_expand(graph_embedding, W_exp, b_exp.reshape(1, -1))
    x0 = jnp.pad(x0f.reshape(N, H0), ((0, NPAD - N), (0, 0)))
    y0, dis = _tc_prep(x0, degp[0].reshape(NPAD, 1), degp[1].reshape(NPAD, 1))
    agg1 = _sc_edge_agg(y0, s2, d2, w2, H0)
    w2p = jnp.pad(W2, ((0, 16 - OUT_DIM), (0, 0)))
    y2 = _tc_mid(agg1[0], agg1[1], y0, dis, W1, b1.reshape(1, -1),
                 gamma.reshape(1, -1), beta.reshape(1, -1),
                 running_mean.reshape(1, -1), running_var.reshape(1, -1), w2p)
    agg2 = _sc_edge_agg(y2, s2, d2, w2, 16)
    b2p = jnp.pad(b2, (0, 16 - OUT_DIM)).reshape(1, 16)
    return _tc_final(agg2[0], agg2[1], y2, dis, b2p)


# speedup vs baseline: 60.7580x; 1.6754x over previous
import functools

import jax
import jax.numpy as jnp
from jax import lax
from jax.experimental import pallas as pl
from jax.experimental.pallas import tpu as pltpu
from jax.experimental.pallas import tpu_sc as plsc

B = 100
EMB = 16
H0 = 32
H1 = 64
OUT_DIM = 3
NUM_NODES = 500
N = B * NUM_NODES
E = 800000

NC = 2
NS = 16
NW = NC * NS
CH = 128
KCH = 2
CPW = 200
NSUP = CPW // KCH
EPAD = NW * CPW * CH
NPAD = 51200
RPT = NPAD // NS
ZR = 400
RB = 2048
GRID = NPAD // RB

_f32 = jnp.float32
_i32 = jnp.int32


def _mesh():
    return plsc.VectorSubcoreMesh(core_axis_name="c", subcore_axis_name="s")


_SC_PARAMS = pltpu.CompilerParams(use_tc_tiling_on_sc=False)


def _sc_deg(d2, w2):
    @functools.partial(
        pl.kernel,
        out_type=jax.ShapeDtypeStruct((NC, NPAD), _f32),
        mesh=_mesh(),
        compiler_params=_SC_PARAMS,
        scratch_types=[
            pltpu.VMEM_SHARED((NPAD,), _f32),
            pltpu.VMEM((RPT,), _f32),
            pltpu.VMEM((3, KCH, CH), _i32),
            pltpu.VMEM((3, KCH, CH), _f32),
            pltpu.SemaphoreType.DMA,
            pltpu.SemaphoreType.DMA,
        ],
    )
    def k(d_hbm, w_hbm, out, acc, zbuf, dbuf, wbuf, sem_e, sem_s):
        c = lax.axis_index("c")
        s = lax.axis_index("s")
        wid = c * NS + s

        def zb(i, _):
            zbuf[pl.ds(i * 16, 16)] = jnp.zeros((16,), _f32)
            return 0

        lax.fori_loop(0, RPT // 16, zb, 0)
        pltpu.sync_copy(zbuf, acc.at[pl.ds(s * RPT, RPT)])
        plsc.subcore_barrier()

        def issue_edge(j, b):
            pltpu.async_copy(d_hbm.at[j], dbuf.at[b], sem_e)
            pltpu.async_copy(w_hbm.at[j], wbuf.at[b], sem_e)

        def wait_edge(b):
            pltpu.make_async_copy(d_hbm.at[0], dbuf.at[b], sem_e).wait()
            pltpu.make_async_copy(w_hbm.at[0], wbuf.at[b], sem_e).wait()

        def issue_scatter(b):
            for kk in range(KCH):
                pltpu.async_copy(
                    wbuf.at[b, kk], acc.at[dbuf.at[b, kk]], sem_s, add=True
                )

        def wait_scatter(b):
            for kk in range(KCH):
                pltpu.make_async_copy(
                    wbuf.at[b, kk], acc.at[dbuf.at[b, kk]], sem_s
                ).wait()

        j0 = wid * NSUP
        issue_edge(j0, 0)
        wait_edge(0)
        issue_scatter(0)
        issue_edge(j0 + 1, 1)
        wait_edge(1)
        issue_scatter(1)
        issue_edge(j0 + 2, 2)

        def body(t, _):
            g = 2 + 3 * t
            for (cur, prv, nxt), dg in (((2, 1, 0), 0), ((0, 2, 1), 1),
                                        ((1, 0, 2), 2)):
                wait_edge(cur)
                wait_scatter(nxt)
                issue_scatter(cur)
                issue_edge(j0 + g + dg + 1, nxt)
            return 0

        lax.fori_loop(0, (NSUP - 4) // 3, body, 0)
        wait_edge(2)
        wait_scatter(0)
        issue_scatter(2)
        issue_edge(j0 + NSUP - 1, 0)
        wait_edge(0)
        wait_scatter(1)
        issue_scatter(0)
        wait_scatter(2)
        wait_scatter(0)
        plsc.subcore_barrier()
        pltpu.sync_copy(acc.at[pl.ds(s * RPT, RPT)], out.at[c, pl.ds(s * RPT, RPT)])

    return k(d2, w2)


def _sc_edge_agg(y, s2, d2, w2, width):
    @functools.partial(
        pl.kernel,
        out_type=jax.ShapeDtypeStruct((NC, NPAD, width), _f32),
        mesh=_mesh(),
        compiler_params=_SC_PARAMS,
        scratch_types=[
            pltpu.VMEM_SHARED((NPAD, width), _f32),
            pltpu.VMEM((3, KCH, CH), _i32),
            pltpu.VMEM((3, KCH, CH), _i32),
            pltpu.VMEM((3, KCH, CH), _f32),
            pltpu.VMEM((3, KCH, CH, width), _f32),
            pltpu.SemaphoreType.DMA,
            pltpu.SemaphoreType.DMA,
            pltpu.SemaphoreType.DMA,
        ],
    )
    def k(y_hbm, s_hbm, d_hbm, w_hbm, out, acc, sbuf, dbuf, wbuf, rows,
          sem_e, sem_g, sem_s):
        c = lax.axis_index("c")
        s = lax.axis_index("s")
        wid = c * NS + s

        def zb(i, _):
            for h in range(width // 16):
                rows[0, 0, i, pl.ds(h * 16, 16)] = jnp.zeros((16,), _f32)
            return 0

        lax.fori_loop(0, CH, zb, 0)
        for r in range(RPT // CH):
            pltpu.sync_copy(rows.at[0, 0], acc.at[pl.ds(s * RPT + r * CH, CH)])
        plsc.subcore_barrier()

        def issue_edge(j, b):
            pltpu.async_copy(s_hbm.at[j], sbuf.at[b], sem_e)
            pltpu.async_copy(d_hbm.at[j], dbuf.at[b], sem_e)
            pltpu.async_copy(w_hbm.at[j], wbuf.at[b], sem_e)

        def wait_edge(b):
            pltpu.make_async_copy(s_hbm.at[0], sbuf.at[b], sem_e).wait()
            pltpu.make_async_copy(d_hbm.at[0], dbuf.at[b], sem_e).wait()
            pltpu.make_async_copy(w_hbm.at[0], wbuf.at[b], sem_e).wait()

        def issue_gather(b):
            for kk in range(KCH):
                pltpu.async_copy(y_hbm.at[sbuf.at[b, kk]], rows.at[b, kk], sem_g)

        def wait_gather(b):
            for kk in range(KCH):
                pltpu.make_async_copy(
                    y_hbm.at[sbuf.at[b, kk]], rows.at[b, kk], sem_g
                ).wait()

        def scale(b):
            for kk in range(KCH):
                def sc_body(q, _):
                    wv16 = wbuf[b, kk, pl.ds(q * 16, 16)]
                    for j2 in range(16):
                        wv = wv16[j2]
                        for h in range(width // 16):
                            rows[b, kk, q * 16 + j2, pl.ds(h * 16, 16)] = (
                                rows[b, kk, q * 16 + j2, pl.ds(h * 16, 16)] * wv
                            )
                    return 0

                lax.fori_loop(0, CH // 16, sc_body, 0)

        def issue_scatter(b):
            for kk in range(KCH):
                pltpu.async_copy(
                    rows.at[b, kk], acc.at[dbuf.at[b, kk]], sem_s, add=True
                )

        def wait_scatter(b):
            for kk in range(KCH):
                pltpu.make_async_copy(
                    rows.at[b, kk], acc.at[dbuf.at[b, kk]], sem_s
                ).wait()

        j0 = wid * NSUP
        issue_edge(j0, 0)
        wait_edge(0)
        issue_gather(0)
        issue_edge(j0 + 1, 1)
        wait_edge(1)
        issue_gather(1)
        issue_edge(j0 + 2, 2)
        wait_gather(0)
        scale(0)
        issue_scatter(0)

        def body(t, _):
            g = 2 + 3 * t
            for (cur, prv, nxt), dg in (((2, 1, 0), 0), ((0, 2, 1), 1),
                                        ((1, 0, 2), 2)):
                wait_edge(cur)
                wait_scatter(nxt)
                issue_gather(cur)
                issue_edge(j0 + g + dg + 1, nxt)
                wait_gather(prv)
                scale(prv)
                issue_scatter(prv)
            return 0

        lax.fori_loop(0, (NSUP - 4) // 3, body, 0)
        wait_edge(2)
        wait_scatter(0)
        issue_gather(2)
        issue_edge(j0 + NSUP - 1, 0)
        wait_gather(1)
        scale(1)
        issue_scatter(1)
        wait_edge(0)
        wait_scatter(1)
        issue_gather(0)
        wait_gather(2)
        scale(2)
        issue_scatter(2)
        wait_gather(0)
        scale(0)
        issue_scatter(0)
        wait_scatter(2)
        wait_scatter(0)
        plsc.subcore_barrier()
        pltpu.sync_copy(
            acc.at[pl.ds(s * RPT, RPT)], out.at[c, pl.ds(s * RPT, RPT), :]
        )

    return k(y, s2, d2, w2)


PR = NPAD * H0 // 128
PRN = N * H0 // 128
MB = 1600
PGRID = PR // MB


def _tc_prep_fused(g, w_exp, b_exp, degp, e_mat):
    def body(g_ref, w_ref, b_ref, d_ref, e_ref, y_ref, dis_ref):
        x0f = lax.dot_general(
            g_ref[...], w_ref[...], (((1,), (1,)), ((), ())),
            preferred_element_type=_f32,
        ) + b_ref[...]
        x0p = x0f.reshape(PRN, 128)
        disp = lax.rsqrt(1.0 + d_ref[0] + d_ref[1])
        dis32 = lax.dot_general(
            disp, e_ref[...], (((1,), (0,)), ((), ())),
            preferred_element_type=_f32,
        ).reshape(PR, 128)
        dis_ref[...] = dis32
        x0full = jnp.concatenate(
            [x0p, jnp.zeros((PR - PRN, 128), _f32)], axis=0)
        y_ref[...] = dis32 * x0full

    return pl.pallas_call(
        body,
        out_shape=[
            jax.ShapeDtypeStruct((PR, 128), _f32),
            jax.ShapeDtypeStruct((PR, 128), _f32),
        ],
    )(g, w_exp, b_exp, degp, e_mat)


def _tc_mid(agg, y0, dis, w1b, b1t, sct, bft, w2b):
    def body(a_ref, y_ref, d_ref, w1_ref, b1_ref, sc_ref, bf_ref, w2_ref,
             o_ref):
        t = d_ref[...] * (a_ref[0] + a_ref[1] + y_ref[...])
        o1 = lax.dot_general(
            t, w1_ref[...], (((1,), (0,)), ((), ())), preferred_element_type=_f32
        ) + b1_ref[...]
        x1 = jnp.maximum(o1 * sc_ref[...] + bf_ref[...], 0.0)
        h2 = lax.dot_general(
            x1, w2_ref[...], (((1,), (0,)), ((), ())), preferred_element_type=_f32
        )
        o_ref[...] = d_ref[...] * h2

    rowspec = pl.BlockSpec((MB, 128), lambda i: (i, 0))
    full = lambda r, w: pl.BlockSpec((r, w), lambda i: (0, 0))
    return pl.pallas_call(
        body,
        grid=(PGRID,),
        in_specs=[
            pl.BlockSpec((NC, MB, 128), lambda i: (0, i, 0)),
            rowspec, rowspec,
            full(128, 256), full(1, 256), full(1, 256), full(1, 256),
            full(256, 128),
        ],
        out_specs=rowspec,
        out_shape=jax.ShapeDtypeStruct((PR, 128), _f32),
    )(agg, y0, dis, w1b, b1t, sct, bft, w2b)


def _tc_final(agg, y2, dis, b2t):
    def body(a_ref, y_ref, d_ref, b_ref, o_ref):
        o_ref[...] = d_ref[...] * (a_ref[0] + a_ref[1] + y_ref[...]) + b_ref[...]

    rowspec = pl.BlockSpec((MB, 128), lambda i: (i, 0))
    return pl.pallas_call(
        body,
        grid=(PGRID,),
        in_specs=[
            pl.BlockSpec((NC, MB, 128), lambda i: (0, i, 0)),
            rowspec, rowspec,
            pl.BlockSpec((1, 128), lambda i: (0, 0)),
        ],
        out_specs=rowspec,
        out_shape=jax.ShapeDtypeStruct((PR, 128), _f32),
    )(agg, y2, dis, b2t)


def _expand_mat():
    cols = jnp.arange(4096)
    j_needed = 4 * (cols // 128) + (cols % 128) // 32
    return (jnp.arange(128)[:, None] == j_needed[None, :]).astype(_f32)


def _blockdiag(wt, bi, bo):
    out = jnp.zeros((4 * bi, 4 * bo), _f32)
    for a in range(4):
        out = out.at[a * bi:(a + 1) * bi, a * bo:(a + 1) * bo].set(wt)
    return out


def kernel(graph_embedding, edge_index, edge_weight, W_exp, b_exp, W1, b1,
           gamma, beta, running_mean, running_var, W2, b2):
    s32 = edge_index[0].astype(_i32)
    d32 = edge_index[1].astype(_i32)
    w = edge_weight.astype(_f32)
    pad = EPAD - E
    esh = (NW * NSUP, KCH, CH)
    padidx = jnp.arange(pad, dtype=_i32) % N
    s2 = jnp.concatenate([s32, padidx]).reshape(esh)
    d2 = jnp.concatenate([d32, padidx]).reshape(esh)
    w2 = jnp.concatenate([w, jnp.zeros((pad,), _f32)]).reshape(esh)

    degp = _sc_deg(d2, w2).reshape(NC, NPAD // 128, 128)
    y0p, dis32 = _tc_prep_fused(graph_embedding, W_exp,
                                b_exp.reshape(1, -1), degp, _expand_mat())
    agg1 = _sc_edge_agg(y0p.reshape(NPAD, H0), s2, d2, w2, H0)
    w1b = _blockdiag(W1.T, H0, H1)
    sc = gamma * lax.rsqrt(running_var + 1e-5)
    b1t = jnp.tile(b1, 4).reshape(1, 256)
    sct = jnp.tile(sc, 4).reshape(1, 256)
    bft = jnp.tile(beta - running_mean * sc, 4).reshape(1, 256)
    w2p32 = jnp.pad(W2, ((0, H0 - OUT_DIM), (0, 0)))
    w2b = _blockdiag(w2p32.T, H1, H0)
    y2p = _tc_mid(agg1.reshape(NC, PR, 128), y0p, dis32,
                  w1b, b1t, sct, bft, w2b)
    agg2 = _sc_edge_agg(y2p.reshape(NPAD, H0), s2, d2, w2, H0)
    b2t = jnp.tile(jnp.pad(b2, (0, H0 - OUT_DIM)), 4).reshape(1, 128)
    o = _tc_final(agg2.reshape(NC, PR, 128), y2p, dis32, b2t)
    return o[:PRN].reshape(N, H0)[:, :OUT_DIM]


# speedup vs baseline: 66.7717x; 1.0990x over previous
import functools

import jax
import jax.numpy as jnp
from jax import lax
from jax.experimental import pallas as pl
from jax.experimental.pallas import tpu as pltpu
from jax.experimental.pallas import tpu_sc as plsc

B = 100
EMB = 16
H0 = 32
H1 = 64
OUT_DIM = 3
NUM_NODES = 500
N = B * NUM_NODES
E = 800000

NC = 2
NS = 16
NW = NC * NS
CH = 128
KCH = 2
CPW = 200
NSUP = CPW // KCH
KD = 8
NSUPD = CPW // KD
EPAD = NW * CPW * CH
NPAD = 51200
RPT = NPAD // NS
ZR = 400
RB = 2048
GRID = NPAD // RB

_f32 = jnp.float32
_i32 = jnp.int32


def _mesh():
    return plsc.VectorSubcoreMesh(core_axis_name="c", subcore_axis_name="s")


_SC_PARAMS = pltpu.CompilerParams(use_tc_tiling_on_sc=False)


def _sc_deg(d2, w2):
    @functools.partial(
        pl.kernel,
        out_type=jax.ShapeDtypeStruct((NC, NPAD), _f32),
        mesh=_mesh(),
        compiler_params=_SC_PARAMS,
        scratch_types=[
            pltpu.VMEM_SHARED((NPAD,), _f32),
            pltpu.VMEM((RPT,), _f32),
            pltpu.VMEM((3, KD, CH), _i32),
            pltpu.VMEM((3, KD, CH), _f32),
            pltpu.SemaphoreType.DMA,
            pltpu.SemaphoreType.DMA,
        ],
    )
    def k(d_hbm, w_hbm, out, acc, zbuf, dbuf, wbuf, sem_e, sem_s):
        c = lax.axis_index("c")
        s = lax.axis_index("s")
        wid = c * NS + s

        def zb(i, _):
            zbuf[pl.ds(i * 16, 16)] = jnp.zeros((16,), _f32)
            return 0

        lax.fori_loop(0, RPT // 16, zb, 0)
        pltpu.sync_copy(zbuf, acc.at[pl.ds(s * RPT, RPT)])
        plsc.subcore_barrier()

        def issue_edge(j, b):
            pltpu.async_copy(d_hbm.at[j], dbuf.at[b], sem_e)
            pltpu.async_copy(w_hbm.at[j], wbuf.at[b], sem_e)

        def wait_edge(b):
            pltpu.make_async_copy(d_hbm.at[0], dbuf.at[b], sem_e).wait()
            pltpu.make_async_copy(w_hbm.at[0], wbuf.at[b], sem_e).wait()

        def issue_scatter(b):
            for kk in range(KD):
                pltpu.async_copy(
                    wbuf.at[b, kk], acc.at[dbuf.at[b, kk]], sem_s, add=True
                )

        def wait_scatter(b):
            for kk in range(KD):
                pltpu.make_async_copy(
                    wbuf.at[b, kk], acc.at[dbuf.at[b, kk]], sem_s
                ).wait()

        j0 = wid * NSUPD
        issue_edge(j0, 0)
        wait_edge(0)
        issue_scatter(0)
        issue_edge(j0 + 1, 1)
        wait_edge(1)
        issue_scatter(1)
        issue_edge(j0 + 2, 2)

        def body(t, _):
            g = 2 + 3 * t
            for (cur, prv, nxt), dg in (((2, 1, 0), 0), ((0, 2, 1), 1),
                                        ((1, 0, 2), 2)):
                wait_edge(cur)
                wait_scatter(nxt)
                issue_scatter(cur)
                issue_edge(j0 + g + dg + 1, nxt)
            return 0

        lax.fori_loop(0, (NSUPD - 4) // 3, body, 0)
        wait_edge(2)
        wait_scatter(0)
        issue_scatter(2)
        issue_edge(j0 + NSUPD - 1, 0)
        wait_edge(0)
        wait_scatter(1)
        issue_scatter(0)
        wait_scatter(2)
        wait_scatter(0)
        plsc.subcore_barrier()
        pltpu.sync_copy(acc.at[pl.ds(s * RPT, RPT)], out.at[c, pl.ds(s * RPT, RPT)])

    return k(d2, w2)


def _sc_edge_agg(y, s2, d2, w2, width):
    @functools.partial(
        pl.kernel,
        out_type=jax.ShapeDtypeStruct((NC, NPAD, width), _f32),
        mesh=_mesh(),
        compiler_params=_SC_PARAMS,
        scratch_types=[
            pltpu.VMEM_SHARED((NPAD, width), _f32),
            pltpu.VMEM((3, KCH, CH), _i32),
            pltpu.VMEM((3, KCH, CH), _i32),
            pltpu.VMEM((3, KCH, CH), _f32),
            pltpu.VMEM((3, KCH, CH, width), _f32),
            pltpu.SemaphoreType.DMA,
            pltpu.SemaphoreType.DMA,
            pltpu.SemaphoreType.DMA,
        ],
    )
    def k(y_hbm, s_hbm, d_hbm, w_hbm, out, acc, sbuf, dbuf, wbuf, rows,
          sem_e, sem_g, sem_s):
        c = lax.axis_index("c")
        s = lax.axis_index("s")
        wid = c * NS + s

        def zb(i, _):
            for h in range(width // 16):
                rows[0, 0, i, pl.ds(h * 16, 16)] = jnp.zeros((16,), _f32)
            return 0

        lax.fori_loop(0, CH, zb, 0)
        for r in range(RPT // CH):
            pltpu.sync_copy(rows.at[0, 0], acc.at[pl.ds(s * RPT + r * CH, CH)])
        plsc.subcore_barrier()

        def issue_edge(j, b):
            pltpu.async_copy(s_hbm.at[j], sbuf.at[b], sem_e)
            pltpu.async_copy(d_hbm.at[j], dbuf.at[b], sem_e)
            pltpu.async_copy(w_hbm.at[j], wbuf.at[b], sem_e)

        def wait_edge(b):
            pltpu.make_async_copy(s_hbm.at[0], sbuf.at[b], sem_e).wait()
            pltpu.make_async_copy(d_hbm.at[0], dbuf.at[b], sem_e).wait()
            pltpu.make_async_copy(w_hbm.at[0], wbuf.at[b], sem_e).wait()

        def issue_gather(b):
            for kk in range(KCH):
                pltpu.async_copy(y_hbm.at[sbuf.at[b, kk]], rows.at[b, kk], sem_g)

        def wait_gather(b):
            for kk in range(KCH):
                pltpu.make_async_copy(
                    y_hbm.at[sbuf.at[b, kk]], rows.at[b, kk], sem_g
                ).wait()

        def scale(b):
            for kk in range(KCH):
                def sc_body(q, _):
                    wv16 = wbuf[b, kk, pl.ds(q * 16, 16)]
                    for j2 in range(16):
                        wv = wv16[j2]
                        for h in range(width // 16):
                            rows[b, kk, q * 16 + j2, pl.ds(h * 16, 16)] = (
                                rows[b, kk, q * 16 + j2, pl.ds(h * 16, 16)] * wv
                            )
                    return 0

                lax.fori_loop(0, CH // 16, sc_body, 0)

        def issue_scatter(b):
            for kk in range(KCH):
                pltpu.async_copy(
                    rows.at[b, kk], acc.at[dbuf.at[b, kk]], sem_s, add=True
                )

        def wait_scatter(b):
            for kk in range(KCH):
                pltpu.make_async_copy(
                    rows.at[b, kk], acc.at[dbuf.at[b, kk]], sem_s
                ).wait()

        j0 = wid * NSUP
        issue_edge(j0, 0)
        wait_edge(0)
        issue_gather(0)
        issue_edge(j0 + 1, 1)
        wait_edge(1)
        issue_gather(1)
        issue_edge(j0 + 2, 2)
        wait_gather(0)
        scale(0)
        issue_scatter(0)

        def body(t, _):
            g = 2 + 3 * t
            for (cur, prv, nxt), dg in (((2, 1, 0), 0), ((0, 2, 1), 1),
                                        ((1, 0, 2), 2)):
                wait_edge(cur)
                wait_scatter(nxt)
                issue_gather(cur)
                issue_edge(j0 + g + dg + 1, nxt)
                wait_gather(prv)
                scale(prv)
                issue_scatter(prv)
            return 0

        lax.fori_loop(0, (NSUP - 4) // 3, body, 0)
        wait_edge(2)
        wait_scatter(0)
        issue_gather(2)
        issue_edge(j0 + NSUP - 1, 0)
        wait_gather(1)
        scale(1)
        issue_scatter(1)
        wait_edge(0)
        wait_scatter(1)
        issue_gather(0)
        wait_gather(2)
        scale(2)
        issue_scatter(2)
        wait_gather(0)
        scale(0)
        issue_scatter(0)
        wait_scatter(2)
        wait_scatter(0)
        plsc.subcore_barrier()
        pltpu.sync_copy(
            acc.at[pl.ds(s * RPT, RPT)], out.at[c, pl.ds(s * RPT, RPT), :]
        )

    return k(y, s2, d2, w2)


PR = NPAD * H0 // 128
PRN = N * H0 // 128
MB = 1600
PGRID = PR // MB


def _tc_expand_pack(g, w_exp, b_exp):
    def body(g_ref, w_ref, b_ref, o_ref):
        x0f = lax.dot_general(
            g_ref[...], w_ref[...], (((1,), (1,)), ((), ())),
            preferred_element_type=_f32,
        ) + b_ref[...]
        x0p = x0f.reshape(PRN, 128)
        o_ref[...] = jnp.concatenate(
            [x0p, jnp.zeros((PR - PRN, 128), _f32)], axis=0)

    return pl.pallas_call(
        body,
        out_shape=jax.ShapeDtypeStruct((PR, 128), _f32),
    )(g, w_exp, b_exp)


def _tc_prep2(degp, e_mat, x0full):
    def body(d_ref, e_ref, x_ref, y_ref, dis_ref):
        disp = lax.rsqrt(1.0 + d_ref[0] + d_ref[1])
        dis32 = lax.dot_general(
            disp, e_ref[...], (((1,), (0,)), ((), ())),
            preferred_element_type=_f32,
        ).reshape(PR, 128)
        dis_ref[...] = dis32
        y_ref[...] = dis32 * x_ref[...]

    return pl.pallas_call(
        body,
        out_shape=[
            jax.ShapeDtypeStruct((PR, 128), _f32),
            jax.ShapeDtypeStruct((PR, 128), _f32),
        ],
    )(degp, e_mat, x0full)


def _tc_mid(agg, y0, dis, w1b, b1t, sct, bft, w2b):
    def body(a_ref, y_ref, d_ref, w1_ref, b1_ref, sc_ref, bf_ref, w2_ref,
             o_ref):
        t = d_ref[...] * (a_ref[0] + a_ref[1] + y_ref[...])
        o1 = lax.dot_general(
            t, w1_ref[...], (((1,), (0,)), ((), ())), preferred_element_type=_f32
        ) + b1_ref[...]
        x1 = jnp.maximum(o1 * sc_ref[...] + bf_ref[...], 0.0)
        h2 = lax.dot_general(
            x1, w2_ref[...], (((1,), (0,)), ((), ())), preferred_element_type=_f32
        )
        o_ref[...] = d_ref[...] * h2

    rowspec = pl.BlockSpec((MB, 128), lambda i: (i, 0))
    full = lambda r, w: pl.BlockSpec((r, w), lambda i: (0, 0))
    return pl.pallas_call(
        body,
        grid=(PGRID,),
        in_specs=[
            pl.BlockSpec((NC, MB, 128), lambda i: (0, i, 0)),
            rowspec, rowspec,
            full(128, 256), full(1, 256), full(1, 256), full(1, 256),
            full(256, 128),
        ],
        out_specs=rowspec,
        out_shape=jax.ShapeDtypeStruct((PR, 128), _f32),
    )(agg, y0, dis, w1b, b1t, sct, bft, w2b)


def _tc_final(agg, y2, dis, b2t):
    def body(a_ref, y_ref, d_ref, b_ref, o_ref):
        o_ref[...] = d_ref[...] * (a_ref[0] + a_ref[1] + y_ref[...]) + b_ref[...]

    rowspec = pl.BlockSpec((MB, 128), lambda i: (i, 0))
    return pl.pallas_call(
        body,
        grid=(PGRID,),
        in_specs=[
            pl.BlockSpec((NC, MB, 128), lambda i: (0, i, 0)),
            rowspec, rowspec,
            pl.BlockSpec((1, 128), lambda i: (0, 0)),
        ],
        out_specs=rowspec,
        out_shape=jax.ShapeDtypeStruct((PR, 128), _f32),
    )(agg, y2, dis, b2t)


def _expand_mat():
    cols = jnp.arange(4096)
    j_needed = 4 * (cols // 128) + (cols % 128) // 32
    return (jnp.arange(128)[:, None] == j_needed[None, :]).astype(_f32)


def _blockdiag(wt, bi, bo):
    out = jnp.zeros((4 * bi, 4 * bo), _f32)
    for a in range(4):
        out = out.at[a * bi:(a + 1) * bi, a * bo:(a + 1) * bo].set(wt)
    return out


def kernel(graph_embedding, edge_index, edge_weight, W_exp, b_exp, W1, b1,
           gamma, beta, running_mean, running_var, W2, b2):
    s32 = edge_index[0].astype(_i32)
    d32 = edge_index[1].astype(_i32)
    w = edge_weight.astype(_f32)
    pad = EPAD - E
    esh = (NW * NSUP, KCH, CH)
    padidx = jnp.arange(pad, dtype=_i32) % N
    s2 = jnp.concatenate([s32, padidx]).reshape(esh)
    d2 = jnp.concatenate([d32, padidx]).reshape(esh)
    w2 = jnp.concatenate([w, jnp.zeros((pad,), _f32)]).reshape(esh)

    d8 = d2.reshape(NW * NSUPD, KD, CH)
    w8 = w2.reshape(NW * NSUPD, KD, CH)
    degp = _sc_deg(d8, w8).reshape(NC, NPAD // 128, 128)
    x0full = _tc_expand_pack(graph_embedding, W_exp, b_exp.reshape(1, -1))
    y0p, dis32 = _tc_prep2(degp, _expand_mat(), x0full)
    agg1 = _sc_edge_agg(y0p.reshape(NPAD, H0), s2, d2, w2, H0)
    w1b = _blockdiag(W1.T, H0, H1)
    sc = gamma * lax.rsqrt(running_var + 1e-5)
    b1t = jnp.tile(b1, 4).reshape(1, 256)
    sct = jnp.tile(sc, 4).reshape(1, 256)
    bft = jnp.tile(beta - running_mean * sc, 4).reshape(1, 256)
    w2p32 = jnp.pad(W2, ((0, H0 - OUT_DIM), (0, 0)))
    w2b = _blockdiag(w2p32.T, H1, H0)
    y2p = _tc_mid(agg1.reshape(NC, PR, 128), y0p, dis32,
                  w1b, b1t, sct, bft, w2b)
    agg2 = _sc_edge_agg(y2p.reshape(NPAD, H0), s2, d2, w2, H0)
    b2t = jnp.tile(jnp.pad(b2, (0, H0 - OUT_DIM)), 4).reshape(1, 128)
    o = _tc_final(agg2.reshape(NC, PR, 128), y2p, dis32, b2t)
    return o[:PRN].reshape(N, H0)[:, :OUT_DIM]


# speedup vs baseline: 67.3631x; 1.0089x over previous
import functools

import jax
import jax.numpy as jnp
from jax import lax
from jax.experimental import pallas as pl
from jax.experimental.pallas import tpu as pltpu
from jax.experimental.pallas import tpu_sc as plsc

B = 100
EMB = 16
H0 = 32
H1 = 64
OUT_DIM = 3
NUM_NODES = 500
N = B * NUM_NODES
E = 800000

NC = 2
NS = 16
NW = NC * NS
CH = 128
KCH = 2
CPW = 200
NSUP = CPW // KCH
KD = 8
NSUPD = CPW // KD
EPAD = NW * CPW * CH
NPAD = 51200
RPT = NPAD // NS
ZR = 400
RB = 2048
GRID = NPAD // RB

_f32 = jnp.float32
_i32 = jnp.int32


def _mesh():
    return plsc.VectorSubcoreMesh(core_axis_name="c", subcore_axis_name="s")


_SC_PARAMS = pltpu.CompilerParams(use_tc_tiling_on_sc=False)


def _sc_deg(d2, w2):
    @functools.partial(
        pl.kernel,
        out_type=jax.ShapeDtypeStruct((NC, NPAD), _f32),
        mesh=_mesh(),
        compiler_params=_SC_PARAMS,
        scratch_types=[
            pltpu.VMEM_SHARED((NPAD,), _f32),
            pltpu.VMEM((RPT,), _f32),
            pltpu.VMEM((3, KD, CH), _i32),
            pltpu.VMEM((3, KD, CH), _f32),
            pltpu.SemaphoreType.DMA,
            pltpu.SemaphoreType.DMA,
        ],
    )
    def k(d_hbm, w_hbm, out, acc, zbuf, dbuf, wbuf, sem_e, sem_s):
        c = lax.axis_index("c")
        s = lax.axis_index("s")
        wid = c * NS + s

        def zb(i, _):
            zbuf[pl.ds(i * 16, 16)] = jnp.zeros((16,), _f32)
            return 0

        lax.fori_loop(0, RPT // 16, zb, 0)
        pltpu.sync_copy(zbuf, acc.at[pl.ds(s * RPT, RPT)])
        plsc.subcore_barrier()

        def issue_edge(j, b):
            pltpu.async_copy(d_hbm.at[j], dbuf.at[b], sem_e)
            pltpu.async_copy(w_hbm.at[j], wbuf.at[b], sem_e)

        def wait_edge(b):
            pltpu.make_async_copy(d_hbm.at[0], dbuf.at[b], sem_e).wait()
            pltpu.make_async_copy(w_hbm.at[0], wbuf.at[b], sem_e).wait()

        def issue_scatter(b):
            for kk in range(KD):
                pltpu.async_copy(
                    wbuf.at[b, kk], acc.at[dbuf.at[b, kk]], sem_s, add=True
                )

        def wait_scatter(b):
            for kk in range(KD):
                pltpu.make_async_copy(
                    wbuf.at[b, kk], acc.at[dbuf.at[b, kk]], sem_s
                ).wait()

        j0 = wid * NSUPD
        issue_edge(j0, 0)
        wait_edge(0)
        issue_scatter(0)
        issue_edge(j0 + 1, 1)
        wait_edge(1)
        issue_scatter(1)
        issue_edge(j0 + 2, 2)

        def body(t, _):
            g = 2 + 3 * t
            for (cur, prv, nxt), dg in (((2, 1, 0), 0), ((0, 2, 1), 1),
                                        ((1, 0, 2), 2)):
                wait_edge(cur)
                wait_scatter(nxt)
                issue_scatter(cur)
                issue_edge(j0 + g + dg + 1, nxt)
            return 0

        lax.fori_loop(0, (NSUPD - 4) // 3, body, 0)
        wait_edge(2)
        wait_scatter(0)
        issue_scatter(2)
        issue_edge(j0 + NSUPD - 1, 0)
        wait_edge(0)
        wait_scatter(1)
        issue_scatter(0)
        wait_scatter(2)
        wait_scatter(0)
        plsc.subcore_barrier()
        pltpu.sync_copy(acc.at[pl.ds(s * RPT, RPT)], out.at[c, pl.ds(s * RPT, RPT)])

    return k(d2, w2)


def _sc_edge_agg(y, s2, d2, w2, width):
    @functools.partial(
        pl.kernel,
        out_type=jax.ShapeDtypeStruct((NC, NPAD, width), _f32),
        mesh=_mesh(),
        compiler_params=_SC_PARAMS,
        scratch_types=[
            pltpu.VMEM_SHARED((NPAD, width), _f32),
            pltpu.VMEM((3, KCH, CH), _i32),
            pltpu.VMEM((3, KCH, CH), _i32),
            pltpu.VMEM((3, KCH, CH), _f32),
            pltpu.VMEM((3, KCH, CH, width), _f32),
            pltpu.SemaphoreType.DMA,
            pltpu.SemaphoreType.DMA,
            pltpu.SemaphoreType.DMA,
        ],
    )
    def k(y_hbm, s_hbm, d_hbm, w_hbm, out, acc, sbuf, dbuf, wbuf, rows,
          sem_e, sem_g, sem_s):
        c = lax.axis_index("c")
        s = lax.axis_index("s")
        wid = c * NS + s

        def zb(i, _):
            for h in range(width // 16):
                rows[0, 0, i, pl.ds(h * 16, 16)] = jnp.zeros((16,), _f32)
            return 0

        lax.fori_loop(0, CH, zb, 0)
        for r in range(RPT // CH):
            pltpu.sync_copy(rows.at[0, 0], acc.at[pl.ds(s * RPT + r * CH, CH)])
        plsc.subcore_barrier()

        def issue_edge(j, b):
            pltpu.async_copy(s_hbm.at[j], sbuf.at[b], sem_e)
            pltpu.async_copy(d_hbm.at[j], dbuf.at[b], sem_e)
            pltpu.async_copy(w_hbm.at[j], wbuf.at[b], sem_e)

        def wait_edge(b):
            pltpu.make_async_copy(s_hbm.at[0], sbuf.at[b], sem_e).wait()
            pltpu.make_async_copy(d_hbm.at[0], dbuf.at[b], sem_e).wait()
            pltpu.make_async_copy(w_hbm.at[0], wbuf.at[b], sem_e).wait()

        def issue_gather(b):
            for kk in range(KCH):
                pltpu.async_copy(y_hbm.at[sbuf.at[b, kk]], rows.at[b, kk], sem_g)

        def wait_gather(b):
            for kk in range(KCH):
                pltpu.make_async_copy(
                    y_hbm.at[sbuf.at[b, kk]], rows.at[b, kk], sem_g
                ).wait()

        def scale(b):
            for kk in range(KCH):
                def sc_body(q, _):
                    wv16 = wbuf[b, kk, pl.ds(q * 16, 16)]
                    for j2 in range(16):
                        wv = wv16[j2]
                        for h in range(width // 16):
                            rows[b, kk, q * 16 + j2, pl.ds(h * 16, 16)] = (
                                rows[b, kk, q * 16 + j2, pl.ds(h * 16, 16)] * wv
                            )
                    return 0

                lax.fori_loop(0, CH // 16, sc_body, 0)

        def issue_scatter(b):
            for kk in range(KCH):
                pltpu.async_copy(
                    rows.at[b, kk], acc.at[dbuf.at[b, kk]], sem_s, add=True
                )

        def wait_scatter(b):
            for kk in range(KCH):
                pltpu.make_async_copy(
                    rows.at[b, kk], acc.at[dbuf.at[b, kk]], sem_s
                ).wait()

        j0 = wid * NSUP
        issue_edge(j0, 0)
        wait_edge(0)
        issue_gather(0)
        issue_edge(j0 + 1, 1)
        wait_edge(1)
        issue_gather(1)
        issue_edge(j0 + 2, 2)
        wait_gather(0)
        scale(0)
        issue_scatter(0)

        def body(t, _):
            g = 2 + 3 * t
            for (cur, prv, nxt), dg in (((2, 1, 0), 0), ((0, 2, 1), 1),
                                        ((1, 0, 2), 2)):
                wait_edge(cur)
                wait_scatter(nxt)
                issue_gather(cur)
                issue_edge(j0 + g + dg + 1, nxt)
                wait_gather(prv)
                scale(prv)
                issue_scatter(prv)
            return 0

        lax.fori_loop(0, (NSUP - 4) // 3, body, 0)
        wait_edge(2)
        wait_scatter(0)
        issue_gather(2)
        issue_edge(j0 + NSUP - 1, 0)
        wait_gather(1)
        scale(1)
        issue_scatter(1)
        wait_edge(0)
        wait_scatter(1)
        issue_gather(0)
        wait_gather(2)
        scale(2)
        issue_scatter(2)
        wait_gather(0)
        scale(0)
        issue_scatter(0)
        wait_scatter(2)
        wait_scatter(0)
        plsc.subcore_barrier()
        pltpu.sync_copy(
            acc.at[pl.ds(s * RPT, RPT)], out.at[c, pl.ds(s * RPT, RPT), :]
        )

    return k(y, s2, d2, w2)


PR = NPAD * H0 // 128
PRN = N * H0 // 128
MB = 1600
PGRID = PR // MB


def _tc_expand_pack(g, w_exp, b_exp):
    def body(g_ref, w_ref, b_ref, o_ref):
        x0f = lax.dot_general(
            g_ref[...], w_ref[...], (((1,), (1,)), ((), ())),
            preferred_element_type=_f32,
        ) + b_ref[...]
        x0p = x0f.reshape(PRN, 128)
        o_ref[...] = jnp.concatenate(
            [x0p, jnp.zeros((PR - PRN, 128), _f32)], axis=0)

    return pl.pallas_call(
        body,
        out_shape=jax.ShapeDtypeStruct((PR, 128), _f32),
    )(g, w_exp, b_exp)


def _tc_prep2(degp, e_mat, e_mat16, x0full):
    def body(d_ref, e_ref, e16_ref, x_ref, y_ref, dis_ref, dpk_ref):
        disp = lax.rsqrt(1.0 + d_ref[0] + d_ref[1])
        dis32 = lax.dot_general(
            disp, e_ref[...], (((1,), (0,)), ((), ())),
            preferred_element_type=_f32,
        ).reshape(PR, 128)
        dis_ref[...] = dis32
        y_ref[...] = dis32 * x_ref[...]
        dis16 = lax.dot_general(
            disp, e16_ref[...], (((1,), (0,)), ((), ())),
            preferred_element_type=_f32,
        )
        dpk_ref[...] = dis16.reshape(PR // 2, 128)

    return pl.pallas_call(
        body,
        out_shape=[
            jax.ShapeDtypeStruct((PR, 128), _f32),
            jax.ShapeDtypeStruct((PR, 128), _f32),
            jax.ShapeDtypeStruct((PR // 2, 128), _f32),
        ],
    )(degp, e_mat, e_mat16, x0full)


def _tc_mid(agg, y0, dis, smat, w1b, b1t, sct, bft, w2b):
    def body(a_ref, y_ref, d_ref, s_ref, w1_ref, b1_ref, sc_ref, bf_ref,
             w2_ref, o_ref):
        d4 = lax.dot_general(
            d_ref[...], s_ref[...], (((1,), (0,)), ((), ())),
            preferred_element_type=_f32,
        )
        t = d_ref[...] * (a_ref[0] + a_ref[1] + y_ref[...])
        o1 = lax.dot_general(
            t, w1_ref[...], (((1,), (0,)), ((), ())), preferred_element_type=_f32
        ) + b1_ref[...]
        x1 = jnp.maximum(o1 * sc_ref[...] + bf_ref[...], 0.0)
        h2 = lax.dot_general(
            x1, w2_ref[...], (((1,), (0,)), ((), ())), preferred_element_type=_f32
        )
        o_ref[...] = d4 * h2

    rowspec = pl.BlockSpec((MB, 128), lambda i: (i, 0))
    full = lambda r, w: pl.BlockSpec((r, w), lambda i: (0, 0))
    return pl.pallas_call(
        body,
        grid=(PGRID,),
        in_specs=[
            pl.BlockSpec((NC, MB, 128), lambda i: (0, i, 0)),
            rowspec, rowspec,
            full(128, 64),
            full(128, 256), full(1, 256), full(1, 256), full(1, 256),
            full(256, 64),
        ],
        out_specs=pl.BlockSpec((MB, 64), lambda i: (i, 0)),
        out_shape=jax.ShapeDtypeStruct((PR, 64), _f32),
    )(agg, y0, dis, smat, w1b, b1t, sct, bft, w2b)


def _tc_final(agg, y2, dis, b2t):
    MB2 = MB // 2
    def body(a_ref, y_ref, d_ref, b_ref, o_ref):
        o_ref[...] = d_ref[...] * (a_ref[0] + a_ref[1] + y_ref[...]) + b_ref[...]

    rowspec = pl.BlockSpec((MB2, 128), lambda i: (i, 0))
    return pl.pallas_call(
        body,
        grid=(PGRID,),
        in_specs=[
            pl.BlockSpec((NC, MB2, 128), lambda i: (0, i, 0)),
            rowspec, rowspec,
            pl.BlockSpec((1, 128), lambda i: (0, 0)),
        ],
        out_specs=rowspec,
        out_shape=jax.ShapeDtypeStruct((PR // 2, 128), _f32),
    )(agg, y2, dis, b2t)


def _expand_mat(width):
    npr = 128 // width
    cols = jnp.arange(128 * width)
    j_needed = npr * (cols // 128) + (cols % 128) // width
    return (jnp.arange(128)[:, None] == j_needed[None, :]).astype(_f32)


def _blockdiag(wt, bi, bo):
    out = jnp.zeros((4 * bi, 4 * bo), _f32)
    for a in range(4):
        out = out.at[a * bi:(a + 1) * bi, a * bo:(a + 1) * bo].set(wt)
    return out


def kernel(graph_embedding, edge_index, edge_weight, W_exp, b_exp, W1, b1,
           gamma, beta, running_mean, running_var, W2, b2):
    s32 = edge_index[0].astype(_i32)
    d32 = edge_index[1].astype(_i32)
    w = edge_weight.astype(_f32)
    pad = EPAD - E
    esh = (NW * NSUP, KCH, CH)
    padidx = jnp.arange(pad, dtype=_i32) % N
    s2 = jnp.concatenate([s32, padidx]).reshape(esh)
    d2 = jnp.concatenate([d32, padidx]).reshape(esh)
    w2 = jnp.concatenate([w, jnp.zeros((pad,), _f32)]).reshape(esh)

    d8 = d2.reshape(NW * NSUPD, KD, CH)
    w8 = w2.reshape(NW * NSUPD, KD, CH)
    degp = _sc_deg(d8, w8).reshape(NC, NPAD // 128, 128)
    x0full = _tc_expand_pack(graph_embedding, W_exp, b_exp.reshape(1, -1))
    y0p, dis32, dis16pk = _tc_prep2(
        degp, _expand_mat(H0), _expand_mat(16), x0full)
    agg1 = _sc_edge_agg(y0p.reshape(NPAD, H0), s2, d2, w2, H0)
    w1b = _blockdiag(W1.T, H0, H1)
    sc = gamma * lax.rsqrt(running_var + 1e-5)
    b1t = jnp.tile(b1, 4).reshape(1, 256)
    sct = jnp.tile(sc, 4).reshape(1, 256)
    bft = jnp.tile(beta - running_mean * sc, 4).reshape(1, 256)
    w2p16 = jnp.pad(W2, ((0, 16 - OUT_DIM), (0, 0)))
    w2b = _blockdiag(w2p16.T, H1, 16)
    lanes = jnp.arange(128)
    smat = ((lanes[:, None] % 32 < 16)
            & (16 * (lanes[:, None] // 32) + lanes[:, None] % 32
               == jnp.arange(64)[None, :])).astype(_f32)
    y2m = _tc_mid(agg1.reshape(NC, PR, 128), y0p, dis32, smat,
                  w1b, b1t, sct, bft, w2b)
    y2pk = y2m.reshape(PR // 2, 128)
    agg2 = _sc_edge_agg(y2pk.reshape(NPAD, 16), s2, d2, w2, 16)
    b2t = jnp.tile(jnp.pad(b2, (0, 16 - OUT_DIM)), 8).reshape(1, 128)
    o = _tc_final(agg2.reshape(NC, PR // 2, 128), y2pk, dis16pk, b2t)
    return o[:N * 16 // 128].reshape(N, 16)[:, :OUT_DIM]
